# Initial kernel scaffold; baseline (speedup 1.0000x reference)
import jax
import jax.numpy as jnp
from jax.experimental import pallas as pl


def kernel(x, edge_index, batch, ggnn_w, w_ih, w_hh, b_ih, b_hh, w_out, b_out):
    raise NotImplementedError("write your pallas kernel here")


# speedup vs baseline: 2.4037x; 2.4037x over previous
import jax
import jax.numpy as jnp
from jax import lax
from jax.experimental import pallas as pl
from jax.experimental.pallas import tpu as pltpu
from jax.experimental.pallas import tpu_sc as plsc

N = 10000
E = 160000
ANN = 256
HID = 64
D = ANN + HID
L = 8
OUT = 256

NC = 2
NS = 16
F = D // NC
CH = 128
NCH = 80
EPT = NCH * CH
EPAD = NS * EPT
NPAD = 10240
RPT = NPAD // NS
RCH = 128
RNCH = RPT // RCH


GRP = 16
NGRP = NCH // GRP


def _sc_scatter_body(m2, src2, dst, zeros, p3, srcbuf, dstbuf, rows, acc, sem):
    cid = lax.axis_index("c")
    sid = lax.axis_index("s")
    row0 = sid * RPT
    pltpu.sync_copy(zeros, acc.at[pl.ds(row0, RPT)])
    plsc.subcore_barrier()

    def group(g, carry):
        base = sid * NCH + g * GRP
        pltpu.sync_copy(src2.at[cid, pl.ds(base, GRP)], srcbuf)
        pltpu.sync_copy(dst.at[pl.ds(base, GRP)], dstbuf)

        def chunk(j, c2):
            pltpu.async_copy(m2.at[srcbuf.at[j]], rows, sem).wait()
            pltpu.sync_copy(rows, acc.at[dstbuf.at[j]], add=True)
            return c2

        lax.fori_loop(0, GRP, chunk, 0, unroll=False)
        return carry

    lax.fori_loop(0, NGRP, group, 0, unroll=False)
    plsc.subcore_barrier()

    def outchunk(k, carry):
        pltpu.sync_copy(acc.at[pl.ds(row0 + k * RCH, RCH)], rows)
        pltpu.sync_copy(rows, p3.at[cid, pl.ds(row0 + k * RCH, RCH)])
        return carry

    lax.fori_loop(0, RNCH, outchunk, 0, unroll=False)


_SC_CACHE = {}


def _sc_scatter(m2, src2, dst3, zeros):
    fn = _SC_CACHE.get("k")
    if fn is None:
        fn = pl.kernel(
            _sc_scatter_body,
            out_type=jax.ShapeDtypeStruct((NC, NPAD, F), jnp.float32),
            mesh=plsc.VectorSubcoreMesh(core_axis_name="c",
                                        subcore_axis_name="s"),
            scratch_types=[
                pltpu.VMEM((GRP, CH), jnp.int32),
                pltpu.VMEM((GRP, CH), jnp.int32),
                pltpu.VMEM((CH, F), jnp.float32),
                pltpu.VMEM_SHARED((NPAD, F), jnp.float32),
                pltpu.SemaphoreType.DMA,
            ],
            compiler_params=pltpu.CompilerParams(use_tc_tiling_on_sc=False),
        )
        _SC_CACHE["k"] = fn
    return fn(m2, src2, dst3, zeros)


BN = 1000
_MM = (((1,), (0,)), ((), ()))
_MT = (((1,), (1,)), ((), ()))


def _m0_body(h_ref, w_ref, m_ref):
    m_ref[...] = lax.dot_general(h_ref[...], w_ref[...], _MM,
                                 preferred_element_type=jnp.float32)


def _m0(h, w):
    return pl.pallas_call(
        _m0_body,
        grid=(N // BN,),
        in_specs=[
            pl.BlockSpec((BN, D), lambda i: (i, 0)),
            pl.BlockSpec((D, D), lambda i: (0, 0)),
        ],
        out_specs=pl.BlockSpec((BN, D), lambda i: (i, 0)),
        out_shape=jax.ShapeDtypeStruct((N, D), jnp.float32),
    )(h, w)


def _gru_body(h_ref, pl_ref, pr_ref,
              wir_ref, wiz_ref, win_ref, whr_ref, whz_ref, whn_ref,
              bi_ref, bh_ref, wnext_ref, out_ref, mn_ref):
    h = h_ref[...]
    aggl = pl_ref[...]
    aggr = pr_ref[...]
    f32 = jnp.float32

    def gi(w_ref):
        w = w_ref[...]
        return (lax.dot_general(aggl, w[:, :F], _MT, preferred_element_type=f32)
                + lax.dot_general(aggr, w[:, F:], _MT, preferred_element_type=f32))

    gi_r = gi(wir_ref) + bi_ref[0, :D][None, :]
    gi_z = gi(wiz_ref) + bi_ref[0, D:2 * D][None, :]
    gi_n = gi(win_ref) + bi_ref[0, 2 * D:][None, :]
    gh_r = (lax.dot_general(h, whr_ref[...], _MT, preferred_element_type=f32)
            + bh_ref[0, :D][None, :])
    gh_z = (lax.dot_general(h, whz_ref[...], _MT, preferred_element_type=f32)
            + bh_ref[0, D:2 * D][None, :])
    gh_n = (lax.dot_general(h, whn_ref[...], _MT, preferred_element_type=f32)
            + bh_ref[0, 2 * D:][None, :])
    r = jax.nn.sigmoid(gi_r + gh_r)
    z = jax.nn.sigmoid(gi_z + gh_z)
    n = jnp.tanh(gi_n + r * gh_n)
    hn = (1.0 - z) * n + z * h
    out_ref[...] = hn
    mn_ref[...] = lax.dot_general(hn, wnext_ref[...], _MM,
                                  preferred_element_type=f32)


def _gru_layer(h, p3, wih, whh, b_ih2, b_hh2, w_next):
    wspec = pl.BlockSpec((D, D), lambda i: (0, 0))
    return pl.pallas_call(
        _gru_body,
        grid=(N // BN,),
        in_specs=[
            pl.BlockSpec((BN, D), lambda i: (i, 0)),
            pl.BlockSpec((BN, F), lambda i: (i, 0)),
            pl.BlockSpec((BN, F), lambda i: (i, 0)),
            wspec, wspec, wspec, wspec, wspec, wspec,
            pl.BlockSpec((1, 3 * D), lambda i: (0, 0)),
            pl.BlockSpec((1, 3 * D), lambda i: (0, 0)),
            wspec,
        ],
        out_specs=[pl.BlockSpec((BN, D), lambda i: (i, 0))] * 2,
        out_shape=[jax.ShapeDtypeStruct((N, D), jnp.float32)] * 2,
    )(h, p3[0], p3[1],
      wih[0], wih[1], wih[2], whh[0], whh[1], whh[2],
      b_ih2, b_hh2, w_next)


def _head_body(h_ref, x_ref, w1_ref, w2_ref, b_ref, out_ref):
    f32 = jnp.float32
    out_ref[...] = (
        lax.dot_general(h_ref[...], w1_ref[...], _MT, preferred_element_type=f32)
        + lax.dot_general(x_ref[...], w2_ref[...], _MT, preferred_element_type=f32)
        + b_ref[0][None, :])


def _head(h, x, w_out, b_out):
    return pl.pallas_call(
        _head_body,
        grid=(N // BN,),
        in_specs=[
            pl.BlockSpec((BN, D), lambda i: (i, 0)),
            pl.BlockSpec((BN, ANN), lambda i: (i, 0)),
            pl.BlockSpec((OUT, D), lambda i: (0, 0)),
            pl.BlockSpec((OUT, ANN), lambda i: (0, 0)),
            pl.BlockSpec((1, OUT), lambda i: (0, 0)),
        ],
        out_specs=pl.BlockSpec((BN, OUT), lambda i: (i, 0)),
        out_shape=jax.ShapeDtypeStruct((N, OUT), jnp.float32),
    )(h, x, w_out[:, :D], w_out[:, D:], b_out[None, :])


def kernel(x, edge_index, batch, ggnn_w, w_ih, w_hh, b_ih, b_hh, w_out, b_out):
    src = edge_index[0]
    dst = edge_index[1]
    srcp = jnp.pad(src, (0, EPAD - E))
    dstp = jnp.pad(dst, (0, EPAD - E), constant_values=N)
    src2 = jnp.stack([2 * srcp, 2 * srcp + 1]).reshape(NC, NS * NCH, CH)
    dst3 = dstp.reshape(NS * NCH, CH)
    zeros = jnp.zeros((RPT, F), jnp.float32)

    wih = (w_ih[:D], w_ih[D:2 * D], w_ih[2 * D:])
    whh = (w_hh[:D], w_hh[D:2 * D], w_hh[2 * D:])
    b_ih2 = b_ih[None, :]
    b_hh2 = b_hh[None, :]

    h = jnp.pad(x, ((0, 0), (0, D - ANN)))
    m = _m0(h, ggnn_w[0])
    for l in range(L):
        p3 = _sc_scatter(m.reshape(NC * N, F), src2, dst3, zeros)[:, :N, :]
        w_next = ggnn_w[(l + 1) % L]
        h, m = _gru_layer(h, p3, wih, whh, b_ih2, b_hh2, w_next)
    return _head(h, x, w_out, b_out)


# speedup vs baseline: 2.8766x; 1.1967x over previous
import jax
import jax.numpy as jnp
from jax import lax
from jax.experimental import pallas as pl
from jax.experimental.pallas import tpu as pltpu
from jax.experimental.pallas import tpu_sc as plsc

N = 10000
E = 160000
ANN = 256
HID = 64
D = ANN + HID
L = 8
OUT = 256

NC = 2
NS = 16
F = D // NC
CH = 64
NCH = 160
EPT = NCH * CH
EPAD = NS * EPT
NPAD = 10240
RPT = NPAD // NS
RCH = 64
RNCH = RPT // RCH


GRP = 32
NGRP = NCH // GRP


def _sc_scatter_body(m2, src2, dst, zeros, p3,
                     srcbuf, dstbuf, rows_a, rows_b, acc, gsa, gsb, ssa, ssb):
    cid = lax.axis_index("c")
    sid = lax.axis_index("s")
    row0 = sid * RPT
    pltpu.sync_copy(zeros, acc.at[pl.ds(row0, RPT)])
    plsc.subcore_barrier()

    def group(g, carry):
        base = sid * NCH + g * GRP
        pltpu.sync_copy(src2.at[cid, pl.ds(base, GRP)], srcbuf)
        pltpu.sync_copy(dst.at[pl.ds(base, GRP)], dstbuf)
        pltpu.async_copy(m2.at[srcbuf.at[0]], rows_a, gsa)

        def pair(t, c2):
            j0 = 2 * t
            j1 = j0 + 1
            jn = (j0 + 2) % GRP
            pltpu.async_copy(m2.at[srcbuf.at[j1]], rows_b, gsb)
            pltpu.make_async_copy(m2.at[srcbuf.at[j0]], rows_a, gsa).wait()
            pltpu.async_copy(rows_a, acc.at[dstbuf.at[j0]], ssa,
                             add=True).wait()
            pltpu.async_copy(m2.at[srcbuf.at[jn]], rows_a, gsa)
            pltpu.make_async_copy(m2.at[srcbuf.at[j1]], rows_b, gsb).wait()
            pltpu.async_copy(rows_b, acc.at[dstbuf.at[j1]], ssb,
                             add=True).wait()
            return c2

        lax.fori_loop(0, GRP // 2, pair, 0, unroll=False)
        pltpu.make_async_copy(m2.at[srcbuf.at[0]], rows_a, gsa).wait()
        return carry

    lax.fori_loop(0, NGRP, group, 0, unroll=False)
    plsc.subcore_barrier()

    def outchunk(k, carry):
        pltpu.sync_copy(acc.at[pl.ds(row0 + k * RCH, RCH)], rows_a)
        pltpu.sync_copy(rows_a, p3.at[cid, pl.ds(row0 + k * RCH, RCH)])
        return carry

    lax.fori_loop(0, RNCH, outchunk, 0, unroll=False)


_SC_CACHE = {}


def _sc_scatter(m2, src2, dst3, zeros):
    fn = _SC_CACHE.get("k")
    if fn is None:
        fn = pl.kernel(
            _sc_scatter_body,
            out_type=jax.ShapeDtypeStruct((NC, NPAD, F), jnp.float32),
            mesh=plsc.VectorSubcoreMesh(core_axis_name="c",
                                        subcore_axis_name="s"),
            scratch_types=[
                pltpu.VMEM((GRP, CH), jnp.int32),
                pltpu.VMEM((GRP, CH), jnp.int32),
                pltpu.VMEM((CH, F), jnp.float32),
                pltpu.VMEM((CH, F), jnp.float32),
                pltpu.VMEM_SHARED((NPAD, F), jnp.float32),
                pltpu.SemaphoreType.DMA,
                pltpu.SemaphoreType.DMA,
                pltpu.SemaphoreType.DMA,
                pltpu.SemaphoreType.DMA,
            ],
            compiler_params=pltpu.CompilerParams(use_tc_tiling_on_sc=False),
        )
        _SC_CACHE["k"] = fn
    return fn(m2, src2, dst3, zeros)


BN = 1000
_MM = (((1,), (0,)), ((), ()))
_MT = (((1,), (1,)), ((), ()))


def _m0_body(h_ref, w_ref, m_ref):
    m_ref[...] = lax.dot_general(h_ref[...], w_ref[...], _MM,
                                 preferred_element_type=jnp.float32)


def _m0(h, w):
    return pl.pallas_call(
        _m0_body,
        grid=(N // BN,),
        in_specs=[
            pl.BlockSpec((BN, D), lambda i: (i, 0)),
            pl.BlockSpec((D, D), lambda i: (0, 0)),
        ],
        out_specs=pl.BlockSpec((BN, D), lambda i: (i, 0)),
        out_shape=jax.ShapeDtypeStruct((N, D), jnp.float32),
    )(h, w)


def _gru_body(h_ref, pl_ref, pr_ref,
              wir_ref, wiz_ref, win_ref, whr_ref, whz_ref, whn_ref,
              bi_ref, bh_ref, wnext_ref, out_ref, mn_ref):
    h = h_ref[...]
    aggl = pl_ref[...]
    aggr = pr_ref[...]
    f32 = jnp.float32

    def gi(w_ref):
        w = w_ref[...]
        return (lax.dot_general(aggl, w[:, :F], _MT, preferred_element_type=f32)
                + lax.dot_general(aggr, w[:, F:], _MT, preferred_element_type=f32))

    gi_r = gi(wir_ref) + bi_ref[0, :D][None, :]
    gi_z = gi(wiz_ref) + bi_ref[0, D:2 * D][None, :]
    gi_n = gi(win_ref) + bi_ref[0, 2 * D:][None, :]
    gh_r = (lax.dot_general(h, whr_ref[...], _MT, preferred_element_type=f32)
            + bh_ref[0, :D][None, :])
    gh_z = (lax.dot_general(h, whz_ref[...], _MT, preferred_element_type=f32)
            + bh_ref[0, D:2 * D][None, :])
    gh_n = (lax.dot_general(h, whn_ref[...], _MT, preferred_element_type=f32)
            + bh_ref[0, 2 * D:][None, :])
    r = jax.nn.sigmoid(gi_r + gh_r)
    z = jax.nn.sigmoid(gi_z + gh_z)
    n = jnp.tanh(gi_n + r * gh_n)
    hn = (1.0 - z) * n + z * h
    out_ref[...] = hn
    mn_ref[...] = lax.dot_general(hn, wnext_ref[...], _MM,
                                  preferred_element_type=f32)


def _gru_layer(h, p3, wih, whh, b_ih2, b_hh2, w_next):
    wspec = pl.BlockSpec((D, D), lambda i: (0, 0))
    return pl.pallas_call(
        _gru_body,
        grid=(N // BN,),
        in_specs=[
            pl.BlockSpec((BN, D), lambda i: (i, 0)),
            pl.BlockSpec((BN, F), lambda i: (i, 0)),
            pl.BlockSpec((BN, F), lambda i: (i, 0)),
            wspec, wspec, wspec, wspec, wspec, wspec,
            pl.BlockSpec((1, 3 * D), lambda i: (0, 0)),
            pl.BlockSpec((1, 3 * D), lambda i: (0, 0)),
            wspec,
        ],
        out_specs=[pl.BlockSpec((BN, D), lambda i: (i, 0))] * 2,
        out_shape=[jax.ShapeDtypeStruct((N, D), jnp.float32)] * 2,
    )(h, p3[0], p3[1],
      wih[0], wih[1], wih[2], whh[0], whh[1], whh[2],
      b_ih2, b_hh2, w_next)


def _head_body(h_ref, x_ref, w1_ref, w2_ref, b_ref, out_ref):
    f32 = jnp.float32
    out_ref[...] = (
        lax.dot_general(h_ref[...], w1_ref[...], _MT, preferred_element_type=f32)
        + lax.dot_general(x_ref[...], w2_ref[...], _MT, preferred_element_type=f32)
        + b_ref[0][None, :])


def _head(h, x, w_out, b_out):
    return pl.pallas_call(
        _head_body,
        grid=(N // BN,),
        in_specs=[
            pl.BlockSpec((BN, D), lambda i: (i, 0)),
            pl.BlockSpec((BN, ANN), lambda i: (i, 0)),
            pl.BlockSpec((OUT, D), lambda i: (0, 0)),
            pl.BlockSpec((OUT, ANN), lambda i: (0, 0)),
            pl.BlockSpec((1, OUT), lambda i: (0, 0)),
        ],
        out_specs=pl.BlockSpec((BN, OUT), lambda i: (i, 0)),
        out_shape=jax.ShapeDtypeStruct((N, OUT), jnp.float32),
    )(h, x, w_out[:, :D], w_out[:, D:], b_out[None, :])


def kernel(x, edge_index, batch, ggnn_w, w_ih, w_hh, b_ih, b_hh, w_out, b_out):
    src = edge_index[0]
    dst = edge_index[1]
    srcp = jnp.pad(src, (0, EPAD - E))
    dstp = jnp.pad(dst, (0, EPAD - E), constant_values=N)
    src2 = jnp.stack([2 * srcp, 2 * srcp + 1]).reshape(NC, NS * NCH, CH)
    dst3 = dstp.reshape(NS * NCH, CH)
    zeros = jnp.zeros((RPT, F), jnp.float32)

    wih = (w_ih[:D], w_ih[D:2 * D], w_ih[2 * D:])
    whh = (w_hh[:D], w_hh[D:2 * D], w_hh[2 * D:])
    b_ih2 = b_ih[None, :]
    b_hh2 = b_hh[None, :]

    h = jnp.pad(x, ((0, 0), (0, D - ANN)))
    m = _m0(h, ggnn_w[0])
    for l in range(L):
        p3 = _sc_scatter(m.reshape(NC * N, F), src2, dst3, zeros)[:, :N, :]
        w_next = ggnn_w[(l + 1) % L]
        h, m = _gru_layer(h, p3, wih, whh, b_ih2, b_hh2, w_next)
    return _head(h, x, w_out, b_out)
